# baseline (speedup 1.0000x reference)
import functools
import jax
import jax.numpy as jnp
from jax import lax
from jax.experimental import pallas as pl
from jax.experimental.pallas import tpu as pltpu
from jax.experimental.pallas import tpu_sc as plsc

EMBED = 32
CHUNK = 128


def _make_sc_gather(NW, C):
    mesh = plsc.VectorSubcoreMesh(core_axis_name="c", subcore_axis_name="s")

    def body(ut_hbm, mt_hbm, uidx_hbm, midx_hbm, ue_hbm, me_hbm,
             uidx_v, midx_v, urows_v, mrows_v, sem):
        wid = lax.axis_index("s") * 2 + lax.axis_index("c")
        pltpu.sync_copy(uidx_hbm.at[wid], uidx_v)
        pltpu.sync_copy(midx_hbm.at[wid], midx_v)
        copies = []
        for j in range(C):
            copies.append(pltpu.async_copy(ut_hbm.at[uidx_v.at[j]], urows_v.at[j], sem))
            copies.append(pltpu.async_copy(mt_hbm.at[midx_v.at[j]], mrows_v.at[j], sem))
        for cp in copies:
            cp.wait()
        pltpu.sync_copy(urows_v, ue_hbm.at[wid])
        pltpu.sync_copy(mrows_v, me_hbm.at[wid])

    return pl.kernel(
        body,
        mesh=mesh,
        compiler_params=pltpu.CompilerParams(use_tc_tiling_on_sc=False),
        out_type=(
            jax.ShapeDtypeStruct((NW, C, CHUNK, EMBED), jnp.float32),
            jax.ShapeDtypeStruct((NW, C, CHUNK, EMBED), jnp.float32),
        ),
        scratch_types=[
            pltpu.VMEM((C, CHUNK), jnp.int32),
            pltpu.VMEM((C, CHUNK), jnp.int32),
            pltpu.VMEM((C, CHUNK, EMBED), jnp.float32),
            pltpu.VMEM((C, CHUNK, EMBED), jnp.float32),
            pltpu.SemaphoreType.DMA,
        ],
    )


def _mlp_body(ue_ref, me_ref, W1_ref, b1_ref, W2_ref, b2_ref, W3_ref, b3_ref, o_ref):
    u = ue_ref[...]
    m = me_ref[...]
    x = (jnp.dot(u, W1_ref[0:EMBED, :], preferred_element_type=jnp.float32)
         + jnp.dot(m, W1_ref[EMBED:2 * EMBED, :], preferred_element_type=jnp.float32)
         + b1_ref[...])
    x = jnp.where(x >= 0, x, 0.01 * x)
    x = jnp.dot(x, W2_ref[...], preferred_element_type=jnp.float32) + b2_ref[...]
    x = jnp.where(x >= 0, x, 0.01 * x)
    o_ref[...] = jnp.dot(x, W3_ref[...], preferred_element_type=jnp.float32) + b3_ref[...]


def _mlp(ue, me, W1, b1, W2, b2, W3, b3, BT):
    B = ue.shape[0]
    grid = (B // BT,)
    return pl.pallas_call(
        _mlp_body,
        grid=grid,
        in_specs=[
            pl.BlockSpec((BT, EMBED), lambda i: (i, 0)),
            pl.BlockSpec((BT, EMBED), lambda i: (i, 0)),
            pl.BlockSpec((2 * EMBED, 128), lambda i: (0, 0)),
            pl.BlockSpec((1, 128), lambda i: (0, 0)),
            pl.BlockSpec((128, 256), lambda i: (0, 0)),
            pl.BlockSpec((1, 256), lambda i: (0, 0)),
            pl.BlockSpec((256, 1), lambda i: (0, 0)),
            pl.BlockSpec((1, 1), lambda i: (0, 0)),
        ],
        out_specs=pl.BlockSpec((BT, 1), lambda i: (i, 0)),
        out_shape=jax.ShapeDtypeStruct((B, 1), jnp.float32),
    )(ue, me, W1, b1.reshape(1, -1), W2, b2.reshape(1, -1), W3, b3.reshape(1, 1))


def kernel(user, movie, user_table, movie_table, W1, b1, W2, b2, W3, b3):
    B = user.shape[0]
    NW = 32
    C = B // (NW * CHUNK)
    uidx = user.astype(jnp.int32).reshape(NW, C, CHUNK)
    midx = movie.astype(jnp.int32).reshape(NW, C, CHUNK)
    ue, me = _make_sc_gather(NW, C)(user_table, movie_table, uidx, midx)
    ue = ue.reshape(B, EMBED)
    me = me.reshape(B, EMBED)
    return _mlp(ue, me, W1, b1, W2, b2, W3, b3, BT=2048)


# speedup vs baseline: 1.0231x; 1.0231x over previous
import functools
import jax
import jax.numpy as jnp
from jax import lax
from jax.scipy.linalg import block_diag as _block_diag
from jax.experimental import pallas as pl
from jax.experimental.pallas import tpu as pltpu
from jax.experimental.pallas import tpu_sc as plsc

EMBED = 32
CHUNK = 128


def _make_sc_gather(NW, C):
    mesh = plsc.VectorSubcoreMesh(core_axis_name="c", subcore_axis_name="s")

    def body(ut_hbm, mt_hbm, uidx_hbm, midx_hbm, ue_hbm, me_hbm,
             uidx_v, midx_v, urows_v, mrows_v, sem):
        wid = lax.axis_index("s") * 2 + lax.axis_index("c")
        pltpu.sync_copy(uidx_hbm.at[wid], uidx_v)
        pltpu.sync_copy(midx_hbm.at[wid], midx_v)
        copies = []
        for j in range(C):
            copies.append(pltpu.async_copy(ut_hbm.at[uidx_v.at[j]], urows_v.at[j], sem))
            copies.append(pltpu.async_copy(mt_hbm.at[midx_v.at[j]], mrows_v.at[j], sem))
        for cp in copies:
            cp.wait()
        pltpu.sync_copy(urows_v, ue_hbm.at[wid])
        pltpu.sync_copy(mrows_v, me_hbm.at[wid])

    return pl.kernel(
        body,
        mesh=mesh,
        compiler_params=pltpu.CompilerParams(use_tc_tiling_on_sc=False),
        out_type=(
            jax.ShapeDtypeStruct((NW, C, CHUNK, EMBED), jnp.float32),
            jax.ShapeDtypeStruct((NW, C, CHUNK, EMBED), jnp.float32),
        ),
        scratch_types=[
            pltpu.VMEM((C, CHUNK), jnp.int32),
            pltpu.VMEM((C, CHUNK), jnp.int32),
            pltpu.VMEM((C, CHUNK, EMBED), jnp.float32),
            pltpu.VMEM((C, CHUNK, EMBED), jnp.float32),
            pltpu.SemaphoreType.DMA,
        ],
    )


def _mlp_body(u4_ref, m4_ref, W1u_ref, W1m_ref, b1_ref, W2_ref, b2_ref,
              W3_ref, b3_ref, o_ref):
    x = (jnp.dot(u4_ref[...], W1u_ref[...], preferred_element_type=jnp.float32)
         + jnp.dot(m4_ref[...], W1m_ref[...], preferred_element_type=jnp.float32)
         + b1_ref[...])
    x = jnp.where(x >= 0, x, 0.01 * x)
    x = jnp.dot(x, W2_ref[...], preferred_element_type=jnp.float32) + b2_ref[...]
    x = jnp.where(x >= 0, x, 0.01 * x)
    o_ref[...] = jnp.dot(x, W3_ref[...], preferred_element_type=jnp.float32) + b3_ref[...]


def _mlp(u4, m4, W1u4, W1m4, b14, W24, b24, W34, b34, BT4):
    B4 = u4.shape[0]
    grid = (B4 // BT4,)
    return pl.pallas_call(
        _mlp_body,
        grid=grid,
        in_specs=[
            pl.BlockSpec((BT4, 128), lambda i: (i, 0)),
            pl.BlockSpec((BT4, 128), lambda i: (i, 0)),
            pl.BlockSpec((128, 512), lambda i: (0, 0)),
            pl.BlockSpec((128, 512), lambda i: (0, 0)),
            pl.BlockSpec((1, 512), lambda i: (0, 0)),
            pl.BlockSpec((512, 1024), lambda i: (0, 0)),
            pl.BlockSpec((1, 1024), lambda i: (0, 0)),
            pl.BlockSpec((1024, 4), lambda i: (0, 0)),
            pl.BlockSpec((1, 4), lambda i: (0, 0)),
        ],
        out_specs=pl.BlockSpec((BT4, 4), lambda i: (i, 0)),
        out_shape=jax.ShapeDtypeStruct((B4, 4), jnp.float32),
    )(u4, m4, W1u4, W1m4, b14.reshape(1, -1), W24, b24.reshape(1, -1),
      W34, b34.reshape(1, -1))


def kernel(user, movie, user_table, movie_table, W1, b1, W2, b2, W3, b3):
    B = user.shape[0]
    NW = 32
    C = B // (NW * CHUNK)
    uidx = user.astype(jnp.int32).reshape(NW, C, CHUNK)
    midx = movie.astype(jnp.int32).reshape(NW, C, CHUNK)
    ue, me = _make_sc_gather(NW, C)(user_table, movie_table, uidx, midx)
    u4 = ue.reshape(B // 4, 128)
    m4 = me.reshape(B // 4, 128)
    W1u4 = _block_diag(*([W1[:EMBED, :]] * 4))
    W1m4 = _block_diag(*([W1[EMBED:, :]] * 4))
    W24 = _block_diag(*([W2] * 4))
    W34 = _block_diag(*([W3] * 4))
    b14 = jnp.tile(b1, 4)
    b24 = jnp.tile(b2, 4)
    b34 = jnp.tile(b3, 4)
    out4 = _mlp(u4, m4, W1u4, W1m4, b14, W24, b24, W34, b34, BT4=512)
    return out4.reshape(B, 1)
